# Initial kernel scaffold; baseline (speedup 1.0000x reference)
import jax
import jax.numpy as jnp
from jax.experimental import pallas as pl


def kernel(x, batch, W1, b1, g1, be1, W2, g_out, be_out):
    raise NotImplementedError("write your pallas kernel here")


# speedup vs baseline: 9.6244x; 9.6244x over previous
import functools

import jax
import jax.numpy as jnp
from jax import lax
from jax.experimental import pallas as pl
from jax.experimental.pallas import tpu as pltpu
from jax.experimental.pallas import tpu_sc as plsc

_N = 320000
_S = 10000
_D = 128
_TEMP = 0.4


_B1 = 1600


def _proj_body(x_ref, w1_ref, b1_ref, g1_ref, be1_ref, w2_ref, y_ref):
    xb = x_ref[...]
    h = jnp.dot(xb, w1_ref[...], preferred_element_type=jnp.float32)
    h = h + b1_ref[...][None, :]
    mu = jnp.mean(h, axis=1, keepdims=True)
    var = jnp.mean((h - mu) ** 2, axis=1, keepdims=True)
    h = (h - mu) / jnp.sqrt(var + 1e-5) * g1_ref[...][None, :] + be1_ref[...][None, :]
    h = 0.5 * h * (1.0 + lax.erf(h * (2.0 ** -0.5)))
    w = jnp.sum(h * w2_ref[...][None, :], axis=1, keepdims=True)
    q = jnp.exp(w / _TEMP)
    y_ref[...] = xb * q


def _project(x, W1, b1, g1, be1, W2):
    grid = _N // _B1
    return pl.pallas_call(
        _proj_body,
        grid=(grid,),
        in_specs=[
            pl.BlockSpec((_B1, _D), lambda i: (i, 0)),
            pl.BlockSpec((_D, _D), lambda i: (0, 0)),
            pl.BlockSpec((_D,), lambda i: (0,)),
            pl.BlockSpec((_D,), lambda i: (0,)),
            pl.BlockSpec((_D,), lambda i: (0,)),
            pl.BlockSpec((_D,), lambda i: (0,)),
        ],
        out_specs=pl.BlockSpec((_B1, _D), lambda i: (i, 0)),
        out_shape=jax.ShapeDtypeStruct((_N, _D), jnp.float32),
    )(x, W1, b1, g1, be1, W2.reshape(_D))


_NC = 2
_NS = 16
_RPS = _N // _NC
_RPT = _RPS // _NS
_R = 80
_NCH = _RPT // _R
_SP = 10240
_SPT = _SP // _NS


def _scatter_body(y_hbm, b_hbm, acc_hbm, acc_sh, ybuf, ibuf, zbuf):
    c = lax.axis_index("c")
    s = lax.axis_index("s")

    zero16 = jnp.zeros((16,), jnp.float32)

    def _zb(i, _):
        zbuf[i // 8, pl.ds((i % 8) * 16, 16)] = zero16
        return 0

    lax.fori_loop(0, 128 * 8, _zb, 0)

    for k in range(_SPT // 128):
        pltpu.sync_copy(zbuf, acc_sh.at[pl.ds(s * _SPT + k * 128, 128)])
    plsc.subcore_barrier()

    base = c * _RPS + s * _RPT

    def _chunk(j, _):
        st = base + j * _R
        pltpu.sync_copy(y_hbm.at[pl.ds(st, _R)], ybuf)
        pltpu.sync_copy(b_hbm.at[pl.ds(st, _R)], ibuf)
        pltpu.sync_copy(ybuf, acc_sh.at[ibuf], add=True)
        return 0

    lax.fori_loop(0, _NCH, _chunk, 0)
    plsc.subcore_barrier()

    for k in range(_SPT // 128):
        r0 = s * _SPT + k * 128
        pltpu.sync_copy(acc_sh.at[pl.ds(r0, 128)], acc_hbm.at[c, pl.ds(r0, 128)])


def _scatter(y, batch):
    mesh = plsc.VectorSubcoreMesh(core_axis_name="c", subcore_axis_name="s")
    f = functools.partial(
        pl.kernel,
        mesh=mesh,
        out_type=jax.ShapeDtypeStruct((_NC, _SP, _D), jnp.float32),
        scratch_types=[
            pltpu.VMEM_SHARED((_SP, _D), jnp.float32),
            pltpu.VMEM((_R, _D), jnp.float32),
            pltpu.VMEM((_R,), jnp.int32),
            pltpu.VMEM((128, _D), jnp.float32),
        ],
        compiler_params=pltpu.CompilerParams(use_tc_tiling_on_sc=False),
    )(_scatter_body)
    return f(y, batch)


_B3 = 1000


def _final_body(a0_ref, a1_ref, g_ref, b_ref, o_ref):
    o = a0_ref[...] + a1_ref[...]
    mu = jnp.mean(o, axis=1, keepdims=True)
    var = jnp.mean((o - mu) ** 2, axis=1, keepdims=True)
    o_ref[...] = (o - mu) / jnp.sqrt(var + 1e-5) * g_ref[...][None, :] + b_ref[...][None, :]


def _finalize(acc, g_out, be_out):
    grid = _S // _B3
    return pl.pallas_call(
        _final_body,
        grid=(grid,),
        in_specs=[
            pl.BlockSpec((_B3, _D), lambda i: (i, 0)),
            pl.BlockSpec((_B3, _D), lambda i: (i, 0)),
            pl.BlockSpec((_D,), lambda i: (0,)),
            pl.BlockSpec((_D,), lambda i: (0,)),
        ],
        out_specs=pl.BlockSpec((_B3, _D), lambda i: (i, 0)),
        out_shape=jax.ShapeDtypeStruct((_S, _D), jnp.float32),
    )(acc[0], acc[1], g_out, be_out)


def kernel(x, batch, W1, b1, g1, be1, W2, g_out, be_out):
    y = _project(x, W1, b1, g1, be1, W2)
    acc = _scatter(y, batch)
    return _finalize(acc, g_out, be_out)


# speedup vs baseline: 13.0972x; 1.3608x over previous
import functools

import jax
import jax.numpy as jnp
from jax import lax
from jax.experimental import pallas as pl
from jax.experimental.pallas import tpu as pltpu
from jax.experimental.pallas import tpu_sc as plsc

_N = 320000
_S = 10000
_D = 128
_TEMP = 0.4


_B1 = 1600


def _proj_body(x_ref, w1_ref, b1_ref, g1_ref, be1_ref, w2_ref, y_ref):
    xb = x_ref[...]
    h = jnp.dot(xb, w1_ref[...], preferred_element_type=jnp.float32)
    h = h + b1_ref[...][None, :]
    mu = jnp.mean(h, axis=1, keepdims=True)
    var = jnp.mean((h - mu) ** 2, axis=1, keepdims=True)
    h = (h - mu) / jnp.sqrt(var + 1e-5) * g1_ref[...][None, :] + be1_ref[...][None, :]
    h = 0.5 * h * (1.0 + lax.erf(h * (2.0 ** -0.5)))
    w = jnp.sum(h * w2_ref[...][None, :], axis=1, keepdims=True)
    q = jnp.exp(w / _TEMP)
    y_ref[...] = xb * q


def _project(x, W1, b1, g1, be1, W2):
    grid = _N // _B1
    return pl.pallas_call(
        _proj_body,
        grid=(grid,),
        in_specs=[
            pl.BlockSpec((_B1, _D), lambda i: (i, 0)),
            pl.BlockSpec((_D, _D), lambda i: (0, 0)),
            pl.BlockSpec((_D,), lambda i: (0,)),
            pl.BlockSpec((_D,), lambda i: (0,)),
            pl.BlockSpec((_D,), lambda i: (0,)),
            pl.BlockSpec((_D,), lambda i: (0,)),
        ],
        out_specs=pl.BlockSpec((_B1, _D), lambda i: (i, 0)),
        out_shape=jax.ShapeDtypeStruct((_N, _D), jnp.float32),
    )(x, W1, b1, g1, be1, W2.reshape(_D))


_NC = 2
_NS = 16
_RPS = _N // _NC
_RPT = _RPS // _NS
_R = 80
_NCH = _RPT // _R
_SP = 10240
_SPT = _SP // _NS


_NBUF = 4


def _scatter_body(y_hbm, b_hbm, acc_hbm, acc_sh,
                  ybufs, ibufs, lsy, lsi, lss, dsem):
    c = lax.axis_index("c")
    s = lax.axis_index("s")
    base = c * _RPS + s * _RPT

    def _start_load(b, j):
        st = base + j * _R
        pltpu.async_copy(y_hbm.at[pl.ds(st, _R)], ybufs[b], lsy[b])
        pltpu.async_copy(b_hbm.at[pl.ds(st, _R)], ibufs[b], lsi[b])

    def _wait_load(b):
        pltpu.make_async_copy(y_hbm.at[pl.ds(0, _R)], ybufs[b], lsy[b]).wait()
        pltpu.make_async_copy(b_hbm.at[pl.ds(0, _R)], ibufs[b], lsi[b]).wait()

    def _start_scatter(b):
        return pltpu.async_copy(ybufs[b], acc_sh.at[ibufs[b]], lss[b], add=True)

    zero16 = jnp.zeros((16,), jnp.float32)

    def _zb(i, _):
        ybufs[0][i // 8, pl.ds((i % 8) * 16, 16)] = zero16
        return 0

    lax.fori_loop(0, _R * 8, _zb, 0)

    for k in range(_SPT // _R):
        pltpu.sync_copy(ybufs[0], acc_sh.at[pl.ds(s * _SPT + k * _R, _R)])

    for b in range(_NBUF):
        _start_load(b, b)
    plsc.subcore_barrier()

    def _group(g2, _):
        j0 = 4 * g2
        for pair in range(2):
            b0, b1 = 2 * pair, 2 * pair + 1
            _wait_load(b0)
            _wait_load(b1)
            sc0 = _start_scatter(b0)
            sc1 = _start_scatter(b1)
            sc0.wait()
            sc1.wait()
            for b in (b0, b1):
                jj = j0 + _NBUF + pair * 2 + (b - b0)

                @pl.when(jj < _NCH)
                def _():
                    _start_load(b, jj)
        return 0

    lax.fori_loop(0, (_NCH - 1) // _NBUF, _group, 0)
    _wait_load(0)
    _start_scatter(0).wait()
    plsc.subcore_barrier()

    drains = []
    for k in range(_SPT // 128):
        r0 = s * _SPT + k * 128
        drains.append(pltpu.async_copy(
            acc_sh.at[pl.ds(r0, 128)], acc_hbm.at[c, pl.ds(r0, 128)], dsem))
    for d in drains:
        d.wait()


def _scatter(y, batch):
    mesh = plsc.VectorSubcoreMesh(core_axis_name="c", subcore_axis_name="s")
    f = functools.partial(
        pl.kernel,
        mesh=mesh,
        out_type=jax.ShapeDtypeStruct((_NC, _SP, _D), jnp.float32),
        scratch_types=[
            pltpu.VMEM_SHARED((_SP, _D), jnp.float32),
            [pltpu.VMEM((_R, _D), jnp.float32) for _ in range(_NBUF)],
            [pltpu.VMEM((_R,), jnp.int32) for _ in range(_NBUF)],
            [pltpu.SemaphoreType.DMA for _ in range(_NBUF)],
            [pltpu.SemaphoreType.DMA for _ in range(_NBUF)],
            [pltpu.SemaphoreType.DMA for _ in range(_NBUF)],
            pltpu.SemaphoreType.DMA,
        ],
        compiler_params=pltpu.CompilerParams(use_tc_tiling_on_sc=False),
    )(_scatter_body)
    return f(y, batch)


_B3 = 1000


def _final_body(a0_ref, a1_ref, g_ref, b_ref, o_ref):
    o = a0_ref[...] + a1_ref[...]
    mu = jnp.mean(o, axis=1, keepdims=True)
    var = jnp.mean((o - mu) ** 2, axis=1, keepdims=True)
    o_ref[...] = (o - mu) / jnp.sqrt(var + 1e-5) * g_ref[...][None, :] + b_ref[...][None, :]


def _finalize(acc, g_out, be_out):
    grid = _S // _B3
    return pl.pallas_call(
        _final_body,
        grid=(grid,),
        in_specs=[
            pl.BlockSpec((_B3, _D), lambda i: (i, 0)),
            pl.BlockSpec((_B3, _D), lambda i: (i, 0)),
            pl.BlockSpec((_D,), lambda i: (0,)),
            pl.BlockSpec((_D,), lambda i: (0,)),
        ],
        out_specs=pl.BlockSpec((_B3, _D), lambda i: (i, 0)),
        out_shape=jax.ShapeDtypeStruct((_S, _D), jnp.float32),
    )(acc[0], acc[1], g_out, be_out)


def kernel(x, batch, W1, b1, g1, be1, W2, g_out, be_out):
    y = _project(x, W1, b1, g1, be1, W2)
    acc = _scatter(y, batch)
    return _finalize(acc, g_out, be_out)


# speedup vs baseline: 13.7145x; 1.0471x over previous
import functools

import jax
import jax.numpy as jnp
from jax import lax
from jax.experimental import pallas as pl
from jax.experimental.pallas import tpu as pltpu
from jax.experimental.pallas import tpu_sc as plsc

_N = 320000
_S = 10000
_D = 128
_TEMP = 0.4


_B1 = 1600


def _proj_body(x_ref, w1c_ref, b1c_ref, g1_ref, be1_ref, w2_ref, ones_ref,
               y_ref):
    xb = x_ref[...]
    hm = jnp.dot(xb, w1c_ref[...], preferred_element_type=jnp.float32)
    hm = hm + b1c_ref[...][None, :]
    var = jnp.dot(hm * hm, ones_ref[...], preferred_element_type=jnp.float32)
    rstd = lax.rsqrt(var[:, 0:1] * (1.0 / _D) + 1e-5)
    hn = (hm * rstd) * g1_ref[...][None, :] + be1_ref[...][None, :]
    hg = hn * (0.5 + 0.5 * lax.erf(hn * (2.0 ** -0.5)))
    w = jnp.dot(hg, w2_ref[...], preferred_element_type=jnp.float32)
    q = jnp.exp(w[:, 0:1] / _TEMP)
    y_ref[...] = xb * q


def _project(x, W1, b1, g1, be1, W2):
    grid = _N // _B1
    w1c = W1 - jnp.mean(W1, axis=1, keepdims=True)
    b1c = b1 - jnp.mean(b1)
    ones = jnp.ones((_D, 1), jnp.float32)
    return pl.pallas_call(
        _proj_body,
        grid=(grid,),
        in_specs=[
            pl.BlockSpec((_B1, _D), lambda i: (i, 0)),
            pl.BlockSpec((_D, _D), lambda i: (0, 0)),
            pl.BlockSpec((_D,), lambda i: (0,)),
            pl.BlockSpec((_D,), lambda i: (0,)),
            pl.BlockSpec((_D,), lambda i: (0,)),
            pl.BlockSpec((_D, 1), lambda i: (0, 0)),
            pl.BlockSpec((_D, 1), lambda i: (0, 0)),
        ],
        out_specs=pl.BlockSpec((_B1, _D), lambda i: (i, 0)),
        out_shape=jax.ShapeDtypeStruct((_N, _D), jnp.float32),
    )(x, w1c, b1c, g1, be1, W2, ones)


_NC = 2
_NS = 16
_RPS = _N // _NC
_RPT = _RPS // _NS
_R = 80
_NCH = _RPT // _R
_SP = 10240
_SPT = _SP // _NS


_NBUF = 4


def _scatter_body(y_hbm, b_hbm, acc_hbm, acc_sh,
                  ybufs, ibufs, lsy, lsi, lss, dsem):
    c = lax.axis_index("c")
    s = lax.axis_index("s")
    base = c * _RPS + s * _RPT

    def _start_load(b, j):
        st = base + j * _R
        pltpu.async_copy(y_hbm.at[pl.ds(st, _R)], ybufs[b], lsy[b])
        pltpu.async_copy(b_hbm.at[pl.ds(st, _R)], ibufs[b], lsi[b])

    def _wait_load(b):
        pltpu.make_async_copy(y_hbm.at[pl.ds(0, _R)], ybufs[b], lsy[b]).wait()
        pltpu.make_async_copy(b_hbm.at[pl.ds(0, _R)], ibufs[b], lsi[b]).wait()

    def _start_scatter(b):
        return pltpu.async_copy(ybufs[b], acc_sh.at[ibufs[b]], lss[b], add=True)

    zero16 = jnp.zeros((16,), jnp.float32)

    def _zb(i, _):
        ybufs[0][i // 8, pl.ds((i % 8) * 16, 16)] = zero16
        return 0

    lax.fori_loop(0, _R * 8, _zb, 0)

    for k in range(_SPT // _R):
        pltpu.sync_copy(ybufs[0], acc_sh.at[pl.ds(s * _SPT + k * _R, _R)])

    for b in range(_NBUF):
        _start_load(b, b)
    plsc.subcore_barrier()

    def _group(g2, _):
        j0 = 4 * g2
        for pair in range(2):
            b0, b1 = 2 * pair, 2 * pair + 1
            _wait_load(b0)
            _wait_load(b1)
            sc0 = _start_scatter(b0)
            sc1 = _start_scatter(b1)
            sc0.wait()
            sc1.wait()
            for b in (b0, b1):
                jj = j0 + _NBUF + pair * 2 + (b - b0)

                @pl.when(jj < _NCH)
                def _():
                    _start_load(b, jj)
        return 0

    lax.fori_loop(0, (_NCH - 1) // _NBUF, _group, 0)
    _wait_load(0)
    _start_scatter(0).wait()
    plsc.subcore_barrier()

    drains = []
    for k in range(_SPT // 128):
        r0 = s * _SPT + k * 128
        drains.append(pltpu.async_copy(
            acc_sh.at[pl.ds(r0, 128)], acc_hbm.at[c, pl.ds(r0, 128)], dsem))
    for d in drains:
        d.wait()


def _scatter(y, batch):
    mesh = plsc.VectorSubcoreMesh(core_axis_name="c", subcore_axis_name="s")
    f = functools.partial(
        pl.kernel,
        mesh=mesh,
        out_type=jax.ShapeDtypeStruct((_NC, _SP, _D), jnp.float32),
        scratch_types=[
            pltpu.VMEM_SHARED((_SP, _D), jnp.float32),
            [pltpu.VMEM((_R, _D), jnp.float32) for _ in range(_NBUF)],
            [pltpu.VMEM((_R,), jnp.int32) for _ in range(_NBUF)],
            [pltpu.SemaphoreType.DMA for _ in range(_NBUF)],
            [pltpu.SemaphoreType.DMA for _ in range(_NBUF)],
            [pltpu.SemaphoreType.DMA for _ in range(_NBUF)],
            pltpu.SemaphoreType.DMA,
        ],
        compiler_params=pltpu.CompilerParams(use_tc_tiling_on_sc=False),
    )(_scatter_body)
    return f(y, batch)


_B3 = 1000


def _final_body(a0_ref, a1_ref, g_ref, b_ref, o_ref):
    o = a0_ref[...] + a1_ref[...]
    mu = jnp.mean(o, axis=1, keepdims=True)
    var = jnp.mean((o - mu) ** 2, axis=1, keepdims=True)
    o_ref[...] = (o - mu) / jnp.sqrt(var + 1e-5) * g_ref[...][None, :] + b_ref[...][None, :]


def _finalize(acc, g_out, be_out):
    grid = _S // _B3
    return pl.pallas_call(
        _final_body,
        grid=(grid,),
        in_specs=[
            pl.BlockSpec((_B3, _D), lambda i: (i, 0)),
            pl.BlockSpec((_B3, _D), lambda i: (i, 0)),
            pl.BlockSpec((_D,), lambda i: (0,)),
            pl.BlockSpec((_D,), lambda i: (0,)),
        ],
        out_specs=pl.BlockSpec((_B3, _D), lambda i: (i, 0)),
        out_shape=jax.ShapeDtypeStruct((_S, _D), jnp.float32),
    )(acc[0], acc[1], g_out, be_out)


def kernel(x, batch, W1, b1, g1, be1, W2, g_out, be_out):
    y = _project(x, W1, b1, g1, be1, W2)
    acc = _scatter(y, batch)
    return _finalize(acc, g_out, be_out)


# speedup vs baseline: 17.0670x; 1.2445x over previous
import functools

import jax
import jax.numpy as jnp
from jax import lax
from jax.experimental import pallas as pl
from jax.experimental.pallas import tpu as pltpu
from jax.experimental.pallas import tpu_sc as plsc

_N = 320000
_S = 10000
_D = 128
_TEMP = 0.4


_B1 = 4000


def _proj_body(x_ref, w1c_ref, b1c_ref, g1_ref, be1_ref, w2_ref, ones_ref,
               y_ref):
    xb = x_ref[...]
    hm = jnp.dot(xb, w1c_ref[...], preferred_element_type=jnp.float32)
    hm = hm + b1c_ref[...][None, :]
    var = jnp.dot(hm * hm, ones_ref[...], preferred_element_type=jnp.float32)
    rstd = lax.rsqrt(var[:, 0:1] * (1.0 / _D) + 1e-5)
    hn = (hm * rstd) * g1_ref[...][None, :] + be1_ref[...][None, :]
    hg = hn * (0.5 + 0.5 * lax.erf(hn * (2.0 ** -0.5)))
    w = jnp.dot(hg, w2_ref[...], preferred_element_type=jnp.float32)
    q = jnp.exp(w[:, 0:1] / _TEMP)
    y_ref[...] = xb * q


def _project(x, W1, b1, g1, be1, W2):
    grid = _N // _B1
    w1c = W1 - jnp.mean(W1, axis=1, keepdims=True)
    b1c = b1 - jnp.mean(b1)
    ones = jnp.ones((_D, 1), jnp.float32)
    return pl.pallas_call(
        _proj_body,
        grid=(grid,),
        in_specs=[
            pl.BlockSpec((_B1, _D), lambda i: (i, 0)),
            pl.BlockSpec((_D, _D), lambda i: (0, 0)),
            pl.BlockSpec((_D,), lambda i: (0,)),
            pl.BlockSpec((_D,), lambda i: (0,)),
            pl.BlockSpec((_D,), lambda i: (0,)),
            pl.BlockSpec((_D, 1), lambda i: (0, 0)),
            pl.BlockSpec((_D, 1), lambda i: (0, 0)),
        ],
        out_specs=pl.BlockSpec((_B1, _D), lambda i: (i, 0)),
        out_shape=jax.ShapeDtypeStruct((_N, _D), jnp.float32),
    )(x, w1c, b1c, g1, be1, W2, ones)


_NC = 2
_NS = 16
_RPS = _N // _NC
_RPT = _RPS // _NS
_R = 80
_NCH = _RPT // _R
_SP = 10240
_SPT = _SP // _NS


_NBUF = 4


def _scatter_body(y_hbm, b_hbm, acc_hbm, acc_sh,
                  ybufs, ibufs, lsy, lsi, lss, dsem):
    c = lax.axis_index("c")
    s = lax.axis_index("s")
    base = c * _RPS + s * _RPT

    def _start_load(b, j):
        st = base + j * _R
        pltpu.async_copy(y_hbm.at[pl.ds(st, _R)], ybufs[b], lsy[b])
        pltpu.async_copy(b_hbm.at[pl.ds(st, _R)], ibufs[b], lsi[b])

    def _wait_load(b):
        pltpu.make_async_copy(y_hbm.at[pl.ds(0, _R)], ybufs[b], lsy[b]).wait()
        pltpu.make_async_copy(b_hbm.at[pl.ds(0, _R)], ibufs[b], lsi[b]).wait()

    def _start_scatter(b):
        return pltpu.async_copy(ybufs[b], acc_sh.at[ibufs[b]], lss[b], add=True)

    zero16 = jnp.zeros((16,), jnp.float32)

    def _zb(i, _):
        ybufs[0][i // 8, pl.ds((i % 8) * 16, 16)] = zero16
        return 0

    lax.fori_loop(0, _R * 8, _zb, 0)

    for k in range(_SPT // _R):
        pltpu.sync_copy(ybufs[0], acc_sh.at[pl.ds(s * _SPT + k * _R, _R)])

    for b in range(_NBUF):
        _start_load(b, b)
    plsc.subcore_barrier()

    def _group(g2, _):
        j0 = 4 * g2
        for pair in range(2):
            b0, b1 = 2 * pair, 2 * pair + 1
            _wait_load(b0)
            _wait_load(b1)
            sc0 = _start_scatter(b0)
            sc1 = _start_scatter(b1)
            sc0.wait()
            sc1.wait()
            for b in (b0, b1):
                jj = j0 + _NBUF + pair * 2 + (b - b0)

                @pl.when(jj < _NCH)
                def _():
                    _start_load(b, jj)
        return 0

    lax.fori_loop(0, (_NCH - 1) // _NBUF, _group, 0)
    _wait_load(0)
    _start_scatter(0).wait()
    plsc.subcore_barrier()

    drains = []
    for k in range(_SPT // 128):
        r0 = s * _SPT + k * 128
        drains.append(pltpu.async_copy(
            acc_sh.at[pl.ds(r0, 128)], acc_hbm.at[c, pl.ds(r0, 128)], dsem))
    for d in drains:
        d.wait()


def _scatter(y, batch):
    mesh = plsc.VectorSubcoreMesh(core_axis_name="c", subcore_axis_name="s")
    f = functools.partial(
        pl.kernel,
        mesh=mesh,
        out_type=jax.ShapeDtypeStruct((_NC, _SP, _D), jnp.float32),
        scratch_types=[
            pltpu.VMEM_SHARED((_SP, _D), jnp.float32),
            [pltpu.VMEM((_R, _D), jnp.float32) for _ in range(_NBUF)],
            [pltpu.VMEM((_R,), jnp.int32) for _ in range(_NBUF)],
            [pltpu.SemaphoreType.DMA for _ in range(_NBUF)],
            [pltpu.SemaphoreType.DMA for _ in range(_NBUF)],
            [pltpu.SemaphoreType.DMA for _ in range(_NBUF)],
            pltpu.SemaphoreType.DMA,
        ],
        compiler_params=pltpu.CompilerParams(use_tc_tiling_on_sc=False),
    )(_scatter_body)
    return f(y, batch)


_B3 = 1000


def _final_body(a0_ref, a1_ref, g_ref, b_ref, o_ref):
    o = a0_ref[...] + a1_ref[...]
    mu = jnp.mean(o, axis=1, keepdims=True)
    var = jnp.mean((o - mu) ** 2, axis=1, keepdims=True)
    o_ref[...] = (o - mu) / jnp.sqrt(var + 1e-5) * g_ref[...][None, :] + b_ref[...][None, :]


def _finalize(acc, g_out, be_out):
    grid = _S // _B3
    return pl.pallas_call(
        _final_body,
        grid=(grid,),
        in_specs=[
            pl.BlockSpec((_B3, _D), lambda i: (i, 0)),
            pl.BlockSpec((_B3, _D), lambda i: (i, 0)),
            pl.BlockSpec((_D,), lambda i: (0,)),
            pl.BlockSpec((_D,), lambda i: (0,)),
        ],
        out_specs=pl.BlockSpec((_B3, _D), lambda i: (i, 0)),
        out_shape=jax.ShapeDtypeStruct((_S, _D), jnp.float32),
    )(acc[0], acc[1], g_out, be_out)


def kernel(x, batch, W1, b1, g1, be1, W2, g_out, be_out):
    y = _project(x, W1, b1, g1, be1, W2)
    acc = _scatter(y, batch)
    return _finalize(acc, g_out, be_out)


# speedup vs baseline: 18.5058x; 1.0843x over previous
import functools

import jax
import jax.numpy as jnp
from jax import lax
from jax.experimental import pallas as pl
from jax.experimental.pallas import tpu as pltpu
from jax.experimental.pallas import tpu_sc as plsc

_N = 320000
_S = 10000
_D = 128
_TEMP = 0.4


_B1 = 8000


def _proj_body(x_ref, w1c_ref, b1c_ref, g1_ref, be1_ref, w2_ref, ones_ref,
               y_ref):
    xb = x_ref[...]
    hm = jnp.dot(xb, w1c_ref[...], preferred_element_type=jnp.float32)
    hm = hm + b1c_ref[...][None, :]
    var = jnp.dot(hm * hm, ones_ref[...], preferred_element_type=jnp.float32)
    rstd = lax.rsqrt(var[:, 0:1] * (1.0 / _D) + 1e-5)
    hn = (hm * rstd) * g1_ref[...][None, :] + be1_ref[...][None, :]
    hg = hn * (0.5 + 0.5 * lax.erf(hn * (2.0 ** -0.5)))
    w = jnp.dot(hg, w2_ref[...], preferred_element_type=jnp.float32)
    q = jnp.exp(w[:, 0:1] / _TEMP)
    y_ref[...] = xb * q


def _project(x, W1, b1, g1, be1, W2):
    grid = _N // _B1
    w1c = W1 - jnp.mean(W1, axis=1, keepdims=True)
    b1c = b1 - jnp.mean(b1)
    ones = jnp.ones((_D, 1), jnp.float32)
    return pl.pallas_call(
        _proj_body,
        grid=(grid,),
        in_specs=[
            pl.BlockSpec((_B1, _D), lambda i: (i, 0)),
            pl.BlockSpec((_D, _D), lambda i: (0, 0)),
            pl.BlockSpec((_D,), lambda i: (0,)),
            pl.BlockSpec((_D,), lambda i: (0,)),
            pl.BlockSpec((_D,), lambda i: (0,)),
            pl.BlockSpec((_D, 1), lambda i: (0, 0)),
            pl.BlockSpec((_D, 1), lambda i: (0, 0)),
        ],
        out_specs=pl.BlockSpec((_B1, _D), lambda i: (i, 0)),
        out_shape=jax.ShapeDtypeStruct((_N, _D), jnp.float32),
    )(x, w1c, b1c, g1, be1, W2, ones)


_NC = 2
_NS = 16
_RPS = _N // _NC
_RPT = _RPS // _NS
_R = 80
_NCH = _RPT // _R
_SP = 10240
_SPT = _SP // _NS


_NBUF = 4


def _scatter_body(y_hbm, b_hbm, acc_hbm, acc_sh,
                  ybufs, ibufs, lsy, lsi, lss, dsem):
    c = lax.axis_index("c")
    s = lax.axis_index("s")
    base = c * _RPS + s * _RPT

    def _start_load(b, j):
        st = base + j * _R
        pltpu.async_copy(y_hbm.at[pl.ds(st, _R)], ybufs[b], lsy[b])
        pltpu.async_copy(b_hbm.at[pl.ds(st, _R)], ibufs[b], lsi[b])

    def _wait_load(b):
        pltpu.make_async_copy(y_hbm.at[pl.ds(0, _R)], ybufs[b], lsy[b]).wait()
        pltpu.make_async_copy(b_hbm.at[pl.ds(0, _R)], ibufs[b], lsi[b]).wait()

    def _start_scatter(b):
        return pltpu.async_copy(ybufs[b], acc_sh.at[ibufs[b]], lss[b], add=True)

    zero16 = jnp.zeros((16,), jnp.float32)

    def _zb(i, _):
        ybufs[0][i // 8, pl.ds((i % 8) * 16, 16)] = zero16
        return 0

    lax.fori_loop(0, _R * 8, _zb, 0)

    for k in range(_SPT // _R):
        pltpu.sync_copy(ybufs[0], acc_sh.at[pl.ds(s * _SPT + k * _R, _R)])

    for b in range(_NBUF):
        _start_load(b, b)
    plsc.subcore_barrier()

    def _group(g2, _):
        j0 = 4 * g2
        for pair in range(2):
            b0, b1 = 2 * pair, 2 * pair + 1
            _wait_load(b0)
            _wait_load(b1)
            sc0 = _start_scatter(b0)
            sc1 = _start_scatter(b1)
            sc0.wait()
            sc1.wait()
            for b in (b0, b1):
                jj = j0 + _NBUF + pair * 2 + (b - b0)

                @pl.when(jj < _NCH)
                def _():
                    _start_load(b, jj)
        return 0

    lax.fori_loop(0, (_NCH - 1) // _NBUF, _group, 0)
    _wait_load(0)
    _start_scatter(0).wait()
    plsc.subcore_barrier()

    drains = []
    for k in range(_SPT // 128):
        r0 = s * _SPT + k * 128
        drains.append(pltpu.async_copy(
            acc_sh.at[pl.ds(r0, 128)], acc_hbm.at[c, pl.ds(r0, 128)], dsem))
    for d in drains:
        d.wait()


def _scatter(y, batch):
    mesh = plsc.VectorSubcoreMesh(core_axis_name="c", subcore_axis_name="s")
    f = functools.partial(
        pl.kernel,
        mesh=mesh,
        out_type=jax.ShapeDtypeStruct((_NC, _SP, _D), jnp.float32),
        scratch_types=[
            pltpu.VMEM_SHARED((_SP, _D), jnp.float32),
            [pltpu.VMEM((_R, _D), jnp.float32) for _ in range(_NBUF)],
            [pltpu.VMEM((_R,), jnp.int32) for _ in range(_NBUF)],
            [pltpu.SemaphoreType.DMA for _ in range(_NBUF)],
            [pltpu.SemaphoreType.DMA for _ in range(_NBUF)],
            [pltpu.SemaphoreType.DMA for _ in range(_NBUF)],
            pltpu.SemaphoreType.DMA,
        ],
        compiler_params=pltpu.CompilerParams(use_tc_tiling_on_sc=False),
    )(_scatter_body)
    return f(y, batch)


_B3 = 1000


def _final_body(a0_ref, a1_ref, g_ref, b_ref, o_ref):
    o = a0_ref[...] + a1_ref[...]
    mu = jnp.mean(o, axis=1, keepdims=True)
    var = jnp.mean((o - mu) ** 2, axis=1, keepdims=True)
    o_ref[...] = (o - mu) / jnp.sqrt(var + 1e-5) * g_ref[...][None, :] + b_ref[...][None, :]


def _finalize(acc, g_out, be_out):
    grid = _S // _B3
    return pl.pallas_call(
        _final_body,
        grid=(grid,),
        in_specs=[
            pl.BlockSpec((_B3, _D), lambda i: (i, 0)),
            pl.BlockSpec((_B3, _D), lambda i: (i, 0)),
            pl.BlockSpec((_D,), lambda i: (0,)),
            pl.BlockSpec((_D,), lambda i: (0,)),
        ],
        out_specs=pl.BlockSpec((_B3, _D), lambda i: (i, 0)),
        out_shape=jax.ShapeDtypeStruct((_S, _D), jnp.float32),
    )(acc[0], acc[1], g_out, be_out)


def kernel(x, batch, W1, b1, g1, be1, W2, g_out, be_out):
    y = _project(x, W1, b1, g1, be1, W2)
    acc = _scatter(y, batch)
    return _finalize(acc, g_out, be_out)


# speedup vs baseline: 18.9438x; 1.0237x over previous
import functools

import jax
import jax.numpy as jnp
from jax import lax
from jax.experimental import pallas as pl
from jax.experimental.pallas import tpu as pltpu
from jax.experimental.pallas import tpu_sc as plsc

_N = 320000
_S = 10000
_D = 128
_TEMP = 0.4


_B1 = 16000


def _proj_body(x_ref, w1c_ref, b1c_ref, g1_ref, be1_ref, w2_ref, ones_ref,
               y_ref):
    xb = x_ref[...]
    hm = jnp.dot(xb, w1c_ref[...], preferred_element_type=jnp.float32)
    hm = hm + b1c_ref[...][None, :]
    var = jnp.dot(hm * hm, ones_ref[...], preferred_element_type=jnp.float32)
    rstd = lax.rsqrt(var[:, 0:1] * (1.0 / _D) + 1e-5)
    hn = (hm * rstd) * g1_ref[...][None, :] + be1_ref[...][None, :]
    hg = hn * (0.5 + 0.5 * lax.erf(hn * (2.0 ** -0.5)))
    w = jnp.dot(hg, w2_ref[...], preferred_element_type=jnp.float32)
    q = jnp.exp(w[:, 0:1] / _TEMP)
    y_ref[...] = xb * q


def _project(x, W1, b1, g1, be1, W2):
    grid = _N // _B1
    w1c = W1 - jnp.mean(W1, axis=1, keepdims=True)
    b1c = b1 - jnp.mean(b1)
    ones = jnp.ones((_D, 1), jnp.float32)
    return pl.pallas_call(
        _proj_body,
        grid=(grid,),
        in_specs=[
            pl.BlockSpec((_B1, _D), lambda i: (i, 0)),
            pl.BlockSpec((_D, _D), lambda i: (0, 0)),
            pl.BlockSpec((_D,), lambda i: (0,)),
            pl.BlockSpec((_D,), lambda i: (0,)),
            pl.BlockSpec((_D,), lambda i: (0,)),
            pl.BlockSpec((_D, 1), lambda i: (0, 0)),
            pl.BlockSpec((_D, 1), lambda i: (0, 0)),
        ],
        out_specs=pl.BlockSpec((_B1, _D), lambda i: (i, 0)),
        out_shape=jax.ShapeDtypeStruct((_N, _D), jnp.float32),
    )(x, w1c, b1c, g1, be1, W2, ones)


_NC = 2
_NS = 16
_RPS = _N // _NC
_RPT = _RPS // _NS
_R = 80
_NCH = _RPT // _R
_SP = 10240
_SPT = _SP // _NS


_NBUF = 4


def _scatter_body(y_hbm, b_hbm, acc_hbm, acc_sh,
                  ybufs, ibufs, lsy, lsi, lss, dsem):
    c = lax.axis_index("c")
    s = lax.axis_index("s")
    base = c * _RPS + s * _RPT

    def _start_load(b, j):
        st = base + j * _R
        pltpu.async_copy(y_hbm.at[pl.ds(st, _R)], ybufs[b], lsy[b])
        pltpu.async_copy(b_hbm.at[pl.ds(st, _R)], ibufs[b], lsi[b])

    def _wait_load(b):
        pltpu.make_async_copy(y_hbm.at[pl.ds(0, _R)], ybufs[b], lsy[b]).wait()
        pltpu.make_async_copy(b_hbm.at[pl.ds(0, _R)], ibufs[b], lsi[b]).wait()

    def _start_scatter(b):
        return pltpu.async_copy(ybufs[b], acc_sh.at[ibufs[b]], lss[b], add=True)

    zero16 = jnp.zeros((16,), jnp.float32)

    def _zb(i, _):
        ybufs[0][i // 8, pl.ds((i % 8) * 16, 16)] = zero16
        return 0

    lax.fori_loop(0, _R * 8, _zb, 0)

    for k in range(_SPT // _R):
        pltpu.sync_copy(ybufs[0], acc_sh.at[pl.ds(s * _SPT + k * _R, _R)])

    for b in range(_NBUF):
        _start_load(b, b)
    plsc.subcore_barrier()

    def _group(g2, _):
        j0 = 4 * g2
        for pair in range(2):
            b0, b1 = 2 * pair, 2 * pair + 1
            _wait_load(b0)
            _wait_load(b1)
            sc0 = _start_scatter(b0)
            sc1 = _start_scatter(b1)
            sc0.wait()
            sc1.wait()
            for b in (b0, b1):
                jj = j0 + _NBUF + pair * 2 + (b - b0)

                @pl.when(jj < _NCH)
                def _():
                    _start_load(b, jj)
        return 0

    lax.fori_loop(0, (_NCH - 1) // _NBUF, _group, 0)
    _wait_load(0)
    _start_scatter(0).wait()
    plsc.subcore_barrier()

    drains = []
    for k in range(_SPT // 128):
        r0 = s * _SPT + k * 128
        drains.append(pltpu.async_copy(
            acc_sh.at[pl.ds(r0, 128)], acc_hbm.at[c, pl.ds(r0, 128)], dsem))
    for d in drains:
        d.wait()


def _scatter(y, batch):
    mesh = plsc.VectorSubcoreMesh(core_axis_name="c", subcore_axis_name="s")
    f = functools.partial(
        pl.kernel,
        mesh=mesh,
        out_type=jax.ShapeDtypeStruct((_NC, _SP, _D), jnp.float32),
        scratch_types=[
            pltpu.VMEM_SHARED((_SP, _D), jnp.float32),
            [pltpu.VMEM((_R, _D), jnp.float32) for _ in range(_NBUF)],
            [pltpu.VMEM((_R,), jnp.int32) for _ in range(_NBUF)],
            [pltpu.SemaphoreType.DMA for _ in range(_NBUF)],
            [pltpu.SemaphoreType.DMA for _ in range(_NBUF)],
            [pltpu.SemaphoreType.DMA for _ in range(_NBUF)],
            pltpu.SemaphoreType.DMA,
        ],
        compiler_params=pltpu.CompilerParams(use_tc_tiling_on_sc=False),
    )(_scatter_body)
    return f(y, batch)


_B3 = 1000


def _final_body(a0_ref, a1_ref, g_ref, b_ref, o_ref):
    o = a0_ref[...] + a1_ref[...]
    mu = jnp.mean(o, axis=1, keepdims=True)
    var = jnp.mean((o - mu) ** 2, axis=1, keepdims=True)
    o_ref[...] = (o - mu) / jnp.sqrt(var + 1e-5) * g_ref[...][None, :] + b_ref[...][None, :]


def _finalize(acc, g_out, be_out):
    grid = _S // _B3
    return pl.pallas_call(
        _final_body,
        grid=(grid,),
        in_specs=[
            pl.BlockSpec((_B3, _D), lambda i: (i, 0)),
            pl.BlockSpec((_B3, _D), lambda i: (i, 0)),
            pl.BlockSpec((_D,), lambda i: (0,)),
            pl.BlockSpec((_D,), lambda i: (0,)),
        ],
        out_specs=pl.BlockSpec((_B3, _D), lambda i: (i, 0)),
        out_shape=jax.ShapeDtypeStruct((_S, _D), jnp.float32),
    )(acc[0], acc[1], g_out, be_out)


def kernel(x, batch, W1, b1, g1, be1, W2, g_out, be_out):
    y = _project(x, W1, b1, g1, be1, W2)
    acc = _scatter(y, batch)
    return _finalize(acc, g_out, be_out)


# speedup vs baseline: 20.2885x; 1.0710x over previous
import functools

import jax
import jax.numpy as jnp
from jax import lax
from jax.experimental import pallas as pl
from jax.experimental.pallas import tpu as pltpu
from jax.experimental.pallas import tpu_sc as plsc

_N = 320000
_S = 10000
_D = 128
_TEMP = 0.4

_P = 2
_HP = _N // _P


_B1 = 16000


def _proj_body(x_ref, w1c_ref, b1c_ref, g1_ref, be1_ref, w2_ref, ones_ref,
               y_ref):
    xb = x_ref[...]
    hm = jnp.dot(xb, w1c_ref[...], preferred_element_type=jnp.float32)
    hm = hm + b1c_ref[...][None, :]
    var = jnp.dot(hm * hm, ones_ref[...], preferred_element_type=jnp.float32)
    rstd = lax.rsqrt(var[:, 0:1] * (1.0 / _D) + 1e-5)
    hn = (hm * rstd) * g1_ref[...][None, :] + be1_ref[...][None, :]
    hg = hn * (0.5 + 0.5 * lax.erf(hn * (2.0 ** -0.5)))
    w = jnp.dot(hg, w2_ref[...], preferred_element_type=jnp.float32)
    q = jnp.exp(w[:, 0:1] / _TEMP)
    y_ref[...] = xb * q


def _project(x, w1c, b1c, g1, be1, W2, ones, part):
    grid = _HP // _B1

    def _xmap(i, p=part):
        return (p * (_HP // _B1) + i, 0)

    return pl.pallas_call(
        _proj_body,
        grid=(grid,),
        in_specs=[
            pl.BlockSpec((_B1, _D), _xmap),
            pl.BlockSpec((_D, _D), lambda i: (0, 0)),
            pl.BlockSpec((_D,), lambda i: (0,)),
            pl.BlockSpec((_D,), lambda i: (0,)),
            pl.BlockSpec((_D,), lambda i: (0,)),
            pl.BlockSpec((_D, 1), lambda i: (0, 0)),
            pl.BlockSpec((_D, 1), lambda i: (0, 0)),
        ],
        out_specs=pl.BlockSpec((_B1, _D), lambda i: (i, 0)),
        out_shape=jax.ShapeDtypeStruct((_HP, _D), jnp.float32),
    )(x, w1c, b1c, g1, be1, W2, ones)


_NC = 2
_NS = 16
_RPS = _HP // _NC
_RPT = _RPS // _NS
_R = 40
_NCH = _RPT // _R
_SP = 10240
_SPT = _SP // _NS

_NBUF = 4


def _scatter_body(y_hbm, b_hbm, acc_hbm, acc_sh,
                  ybufs, ibufs, lsy, lsi, lss, dsem, *, part):
    c = lax.axis_index("c")
    s = lax.axis_index("s")
    gbase = part * _HP + c * _RPS + s * _RPT
    lbase = c * _RPS + s * _RPT

    def _start_load(b, j):
        pltpu.async_copy(y_hbm.at[pl.ds(lbase + j * _R, _R)], ybufs[b], lsy[b])
        pltpu.async_copy(b_hbm.at[pl.ds(gbase + j * _R, _R)], ibufs[b], lsi[b])

    def _wait_load(b):
        pltpu.make_async_copy(y_hbm.at[pl.ds(0, _R)], ybufs[b], lsy[b]).wait()
        pltpu.make_async_copy(b_hbm.at[pl.ds(0, _R)], ibufs[b], lsi[b]).wait()

    def _start_scatter(b):
        return pltpu.async_copy(ybufs[b], acc_sh.at[ibufs[b]], lss[b], add=True)

    zero16 = jnp.zeros((16,), jnp.float32)

    def _zb(i, _):
        ybufs[0][i // 8, pl.ds((i % 8) * 16, 16)] = zero16
        return 0

    lax.fori_loop(0, _R * 8, _zb, 0)

    for k in range(_SPT // _R):
        pltpu.sync_copy(ybufs[0], acc_sh.at[pl.ds(s * _SPT + k * _R, _R)])

    for b in range(_NBUF):
        _start_load(b, b)
    plsc.subcore_barrier()

    def _group(g2, _):
        j0 = 4 * g2
        for pair in range(2):
            b0, b1 = 2 * pair, 2 * pair + 1
            _wait_load(b0)
            _wait_load(b1)
            sc0 = _start_scatter(b0)
            sc1 = _start_scatter(b1)
            sc0.wait()
            sc1.wait()
            for b in (b0, b1):
                jj = j0 + _NBUF + pair * 2 + (b - b0)

                @pl.when(jj < _NCH)
                def _():
                    _start_load(b, jj)
        return 0

    lax.fori_loop(0, (_NCH - 1) // _NBUF, _group, 0)
    _wait_load(0)
    _start_scatter(0).wait()
    plsc.subcore_barrier()

    drains = []
    for k in range(_SPT // 128):
        r0 = s * _SPT + k * 128
        drains.append(pltpu.async_copy(
            acc_sh.at[pl.ds(r0, 128)], acc_hbm.at[c, pl.ds(r0, 128)], dsem))
    for d in drains:
        d.wait()


def _scatter(y, batch, part):
    mesh = plsc.VectorSubcoreMesh(core_axis_name="c", subcore_axis_name="s")
    f = functools.partial(
        pl.kernel,
        mesh=mesh,
        out_type=jax.ShapeDtypeStruct((_NC, _SP, _D), jnp.float32),
        scratch_types=[
            pltpu.VMEM_SHARED((_SP, _D), jnp.float32),
            [pltpu.VMEM((_R, _D), jnp.float32) for _ in range(_NBUF)],
            [pltpu.VMEM((_R,), jnp.int32) for _ in range(_NBUF)],
            [pltpu.SemaphoreType.DMA for _ in range(_NBUF)],
            [pltpu.SemaphoreType.DMA for _ in range(_NBUF)],
            [pltpu.SemaphoreType.DMA for _ in range(_NBUF)],
            pltpu.SemaphoreType.DMA,
        ],
        compiler_params=pltpu.CompilerParams(use_tc_tiling_on_sc=False),
    )(functools.partial(_scatter_body, part=part))
    return f(y, batch)


_B3 = 1000


def _final_body(a0_ref, a1_ref, a2_ref, a3_ref, g_ref, b_ref, o_ref):
    o = (a0_ref[0] + a1_ref[0]) + (a2_ref[0] + a3_ref[0])
    mu = jnp.mean(o, axis=1, keepdims=True)
    var = jnp.mean((o - mu) ** 2, axis=1, keepdims=True)
    o_ref[...] = (o - mu) / jnp.sqrt(var + 1e-5) * g_ref[...][None, :] + b_ref[...][None, :]


def _finalize(accs, g_out, be_out):
    grid = _S // _B3

    def _m0(i):
        return (0, i, 0)

    def _m1(i):
        return (1, i, 0)

    return pl.pallas_call(
        _final_body,
        grid=(grid,),
        in_specs=[pl.BlockSpec((1, _B3, _D), _m0),
                  pl.BlockSpec((1, _B3, _D), _m1),
                  pl.BlockSpec((1, _B3, _D), _m0),
                  pl.BlockSpec((1, _B3, _D), _m1),
                  pl.BlockSpec((_D,), lambda i: (0,)),
                  pl.BlockSpec((_D,), lambda i: (0,))],
        out_specs=pl.BlockSpec((_B3, _D), lambda i: (i, 0)),
        out_shape=jax.ShapeDtypeStruct((_S, _D), jnp.float32),
    )(accs[0], accs[0], accs[1], accs[1], g_out, be_out)


def kernel(x, batch, W1, b1, g1, be1, W2, g_out, be_out):
    w1c = W1 - jnp.mean(W1, axis=1, keepdims=True)
    b1c = b1 - jnp.mean(b1)
    ones = jnp.ones((_D, 1), jnp.float32)
    accs = []
    for p in range(_P):
        y = _project(x, w1c, b1c, g1, be1, W2, ones, p)
        accs.append(_scatter(y, batch, p))
    return _finalize(accs, g_out, be_out)


# speedup vs baseline: 20.5376x; 1.0123x over previous
import functools

import jax
import jax.numpy as jnp
from jax import lax
from jax.experimental import pallas as pl
from jax.experimental.pallas import tpu as pltpu
from jax.experimental.pallas import tpu_sc as plsc

_N = 320000
_S = 10000
_D = 128
_TEMP = 0.4

_P = 2
_HP = _N // _P


_B1 = 16000


def _proj_body(x_ref, w1c_ref, b1c_ref, g1_ref, be1_ref, w2_ref, ones_ref,
               y_ref):
    xb = x_ref[...]
    hm = jnp.dot(xb, w1c_ref[...], preferred_element_type=jnp.float32)
    hm = hm + b1c_ref[...][None, :]
    var = jnp.dot(hm * hm, ones_ref[...], preferred_element_type=jnp.float32)
    rstd = lax.rsqrt(var[:, 0:1] * (1.0 / _D) + 1e-5)
    hn = (hm * rstd) * g1_ref[...][None, :] + be1_ref[...][None, :]
    hg = hn * (0.5 + 0.5 * lax.erf(hn * (2.0 ** -0.5)))
    w = jnp.dot(hg, w2_ref[...], preferred_element_type=jnp.float32)
    q = jnp.exp(w[:, 0:1] / _TEMP)
    y_ref[...] = xb * q


def _project(x, w1c, b1c, g1, be1, W2, ones, part):
    grid = _HP // _B1

    def _xmap(i, p=part):
        return (p * (_HP // _B1) + i, 0)

    return pl.pallas_call(
        _proj_body,
        grid=(grid,),
        in_specs=[
            pl.BlockSpec((_B1, _D), _xmap),
            pl.BlockSpec((_D, _D), lambda i: (0, 0)),
            pl.BlockSpec((_D,), lambda i: (0,)),
            pl.BlockSpec((_D,), lambda i: (0,)),
            pl.BlockSpec((_D,), lambda i: (0,)),
            pl.BlockSpec((_D, 1), lambda i: (0, 0)),
            pl.BlockSpec((_D, 1), lambda i: (0, 0)),
        ],
        out_specs=pl.BlockSpec((_B1, _D), lambda i: (i, 0)),
        out_shape=jax.ShapeDtypeStruct((_HP, _D), jnp.float32),
    )(x, w1c, b1c, g1, be1, W2, ones)


_NC = 2
_NS = 16
_RPS = _HP // _NC
_RPT = _RPS // _NS
_R = 40
_NCH = _RPT // _R
_SP = 10240
_SPT = _SP // _NS

_NBUF = 8


def _scatter_body(y_hbm, b_hbm, acc_hbm, acc_sh,
                  ybufs, ibufs, lsy, lsi, lss, dsem, *, part):
    c = lax.axis_index("c")
    s = lax.axis_index("s")
    gbase = part * _HP + c * _RPS + s * _RPT
    lbase = c * _RPS + s * _RPT

    def _start_load(b, j):
        pltpu.async_copy(y_hbm.at[pl.ds(lbase + j * _R, _R)], ybufs[b], lsy[b])
        pltpu.async_copy(b_hbm.at[pl.ds(gbase + j * _R, _R)], ibufs[b], lsi[b])

    def _wait_load(b):
        pltpu.make_async_copy(y_hbm.at[pl.ds(0, _R)], ybufs[b], lsy[b]).wait()
        pltpu.make_async_copy(b_hbm.at[pl.ds(0, _R)], ibufs[b], lsi[b]).wait()

    def _start_scatter(b):
        return pltpu.async_copy(ybufs[b], acc_sh.at[ibufs[b]], lss[b], add=True)

    zero16 = jnp.zeros((16,), jnp.float32)

    def _zb(i, _):
        ybufs[0][i // 8, pl.ds((i % 8) * 16, 16)] = zero16
        return 0

    lax.fori_loop(0, _R * 8, _zb, 0)

    for k in range(_SPT // _R):
        pltpu.sync_copy(ybufs[0], acc_sh.at[pl.ds(s * _SPT + k * _R, _R)])

    for b in range(_NBUF):
        _start_load(b, b)
    plsc.subcore_barrier()

    def _group(g2, _):
        j0 = 8 * g2
        for half in range(2):
            bs = [4 * half + t for t in range(4)]
            for b in bs:
                _wait_load(b)
            scs = [_start_scatter(b) for b in bs]
            for sc in scs:
                sc.wait()
            for t, b in enumerate(bs):
                _start_load(b, j0 + _NBUF + 4 * half + t)
        return 0

    _TAIL = _NCH % 8
    _MAIN = (_NCH - _TAIL - _NBUF) // 8
    lax.fori_loop(0, _MAIN, _group, 0)
    for half in range(2):
        bs = [4 * half + t for t in range(4)]
        for b in bs:
            _wait_load(b)
        scs = [_start_scatter(b) for b in bs]
        for sc in scs:
            sc.wait()
    for b in range(_TAIL):
        _start_load(b, _NCH - _TAIL + b)
    for b in range(_TAIL):
        _wait_load(b)
    scs = [_start_scatter(b) for b in range(_TAIL)]
    for sc in scs:
        sc.wait()
    plsc.subcore_barrier()

    drains = []
    for k in range(_SPT // 128):
        r0 = s * _SPT + k * 128
        drains.append(pltpu.async_copy(
            acc_sh.at[pl.ds(r0, 128)], acc_hbm.at[c, pl.ds(r0, 128)], dsem))
    for d in drains:
        d.wait()


def _scatter(y, batch, part):
    mesh = plsc.VectorSubcoreMesh(core_axis_name="c", subcore_axis_name="s")
    f = functools.partial(
        pl.kernel,
        mesh=mesh,
        out_type=jax.ShapeDtypeStruct((_NC, _SP, _D), jnp.float32),
        scratch_types=[
            pltpu.VMEM_SHARED((_SP, _D), jnp.float32),
            [pltpu.VMEM((_R, _D), jnp.float32) for _ in range(_NBUF)],
            [pltpu.VMEM((_R,), jnp.int32) for _ in range(_NBUF)],
            [pltpu.SemaphoreType.DMA for _ in range(_NBUF)],
            [pltpu.SemaphoreType.DMA for _ in range(_NBUF)],
            [pltpu.SemaphoreType.DMA for _ in range(_NBUF)],
            pltpu.SemaphoreType.DMA,
        ],
        compiler_params=pltpu.CompilerParams(use_tc_tiling_on_sc=False),
    )(functools.partial(_scatter_body, part=part))
    return f(y, batch)


_B3 = 1000


def _final_body(a0_ref, a1_ref, a2_ref, a3_ref, g_ref, b_ref, o_ref):
    o = (a0_ref[0] + a1_ref[0]) + (a2_ref[0] + a3_ref[0])
    mu = jnp.mean(o, axis=1, keepdims=True)
    var = jnp.mean((o - mu) ** 2, axis=1, keepdims=True)
    o_ref[...] = (o - mu) / jnp.sqrt(var + 1e-5) * g_ref[...][None, :] + b_ref[...][None, :]


def _finalize(accs, g_out, be_out):
    grid = _S // _B3

    def _m0(i):
        return (0, i, 0)

    def _m1(i):
        return (1, i, 0)

    return pl.pallas_call(
        _final_body,
        grid=(grid,),
        in_specs=[pl.BlockSpec((1, _B3, _D), _m0),
                  pl.BlockSpec((1, _B3, _D), _m1),
                  pl.BlockSpec((1, _B3, _D), _m0),
                  pl.BlockSpec((1, _B3, _D), _m1),
                  pl.BlockSpec((_D,), lambda i: (0,)),
                  pl.BlockSpec((_D,), lambda i: (0,))],
        out_specs=pl.BlockSpec((_B3, _D), lambda i: (i, 0)),
        out_shape=jax.ShapeDtypeStruct((_S, _D), jnp.float32),
    )(accs[0], accs[0], accs[1], accs[1], g_out, be_out)


def kernel(x, batch, W1, b1, g1, be1, W2, g_out, be_out):
    w1c = W1 - jnp.mean(W1, axis=1, keepdims=True)
    b1c = b1 - jnp.mean(b1)
    ones = jnp.ones((_D, 1), jnp.float32)
    accs = []
    for p in range(_P):
        y = _project(x, w1c, b1c, g1, be1, W2, ones, p)
        accs.append(_scatter(y, batch, p))
    return _finalize(accs, g_out, be_out)
